# baseline (speedup 1.0000x reference)
import functools

import jax
import jax.numpy as jnp
from jax import lax
from jax.experimental import pallas as pl
from jax.experimental.pallas import tpu as pltpu
from jax.experimental.pallas import tpu_sc as plsc

_LANES = 16


def _make_sc_kernel(B, N, num_workers, chunk):
    num_cores = 1
    mesh = plsc.VectorSubcoreMesh(
        core_axis_name="c", subcore_axis_name="s", num_cores=num_cores
    )
    steps = chunk // _LANES
    half = chunk // 2
    half_steps = steps // 2

    @functools.partial(
        pl.kernel,
        mesh=mesh,
        out_type=jax.ShapeDtypeStruct((B,), jnp.float32),
        scratch_types=[
            pltpu.VMEM((chunk,), jnp.float32),
            pltpu.VMEM((N,), jnp.float32),
            pltpu.VMEM((chunk,), jnp.float32),
            pltpu.SemaphoreType.DMA,
            pltpu.SemaphoreType.DMA,
        ],
        compiler_params=pltpu.CompilerParams(needs_layout_passes=False),
    )
    def sc_kernel(ts_hbm, table_hbm, out_hbm, ts_v, tab_v, out_v, sem_a, sem_b):
        wid = lax.axis_index("s") * num_cores + lax.axis_index("c")
        base = wid * chunk
        cp_tab = pltpu.async_copy(table_hbm, tab_v, sem_a)
        cp_ts = pltpu.async_copy(ts_hbm.at[pl.ds(base, chunk)], ts_v, sem_b)
        cp_tab.wait()
        cp_ts.wait()

        def interp(i):
            t = ts_v[pl.ds(i * _LANES, _LANES)]
            idx = t * jnp.float32(N - 1)
            lo = idx.astype(jnp.int32)
            w = idx - lo.astype(jnp.float32)
            hi = jnp.minimum(lo + 1, N - 1)
            lov = plsc.load_gather(tab_v, [lo])
            hiv = plsc.load_gather(tab_v, [hi])
            out_v[pl.ds(i * _LANES, _LANES)] = lov + w * (hiv - lov)

        plsc.parallel_loop(0, half_steps, 1, unroll=4)(interp)
        cp_out0 = pltpu.async_copy(
            out_v.at[pl.ds(0, half)], out_hbm.at[pl.ds(base, half)], sem_a
        )
        plsc.parallel_loop(half_steps, steps, 1, unroll=4)(interp)
        cp_out1 = pltpu.async_copy(
            out_v.at[pl.ds(half, half)], out_hbm.at[pl.ds(base + half, half)], sem_b
        )
        cp_out0.wait()
        cp_out1.wait()

    return sc_kernel


@jax.jit
def kernel(timesteps, noise_levels):
    B = timesteps.shape[0]
    N = noise_levels.shape[0]
    num_workers = 16
    chunk = B // num_workers
    out = _make_sc_kernel(B, N, num_workers, chunk)(
        timesteps.reshape(B), noise_levels
    )
    return out.reshape(B, 1)


# speedup vs baseline: 1.0021x; 1.0021x over previous
import functools

import jax
import jax.numpy as jnp
from jax import lax
from jax.experimental import pallas as pl
from jax.experimental.pallas import tpu as pltpu
from jax.experimental.pallas import tpu_sc as plsc

_LANES = 16


def _make_sc_kernel(B, N, chunk):
    num_cores = 1
    mesh = plsc.VectorSubcoreMesh(
        core_axis_name="c", subcore_axis_name="s", num_cores=num_cores
    )
    steps = chunk // _LANES

    @functools.partial(
        pl.kernel,
        mesh=mesh,
        out_type=jax.ShapeDtypeStruct((B,), jnp.float32),
        scratch_types=[
            pltpu.VMEM((chunk,), jnp.float32),
            pltpu.VMEM((N,), jnp.float32),
            pltpu.VMEM((chunk,), jnp.float32),
            pltpu.SemaphoreType.DMA,
            pltpu.SemaphoreType.DMA,
        ],
        compiler_params=pltpu.CompilerParams(needs_layout_passes=False),
    )
    def sc_kernel(ts_hbm, table_hbm, out_hbm, ts_v, tab_v, out_v, sem_a, sem_b):
        wid = lax.axis_index("s") * num_cores + lax.axis_index("c")
        base = wid * chunk
        cp_tab = pltpu.async_copy(table_hbm, tab_v, sem_a)
        cp_ts = pltpu.async_copy(ts_hbm.at[pl.ds(base, chunk)], ts_v, sem_b)
        cp_tab.wait()
        cp_ts.wait()

        @plsc.parallel_loop(0, steps, 1, unroll=8)
        def _(i):
            t = ts_v[pl.ds(i * _LANES, _LANES)]
            idx = t * jnp.float32(N - 1)
            lo = idx.astype(jnp.int32)
            w = idx - lo.astype(jnp.float32)
            hi = jnp.minimum(lo + 1, N - 1)
            lov = plsc.load_gather(tab_v, [lo])
            hiv = plsc.load_gather(tab_v, [hi])
            out_v[pl.ds(i * _LANES, _LANES)] = (1.0 - w) * lov + w * hiv

        pltpu.sync_copy(out_v, out_hbm.at[pl.ds(base, chunk)])

    return sc_kernel


@jax.jit
def kernel(timesteps, noise_levels):
    B = timesteps.shape[0]
    N = noise_levels.shape[0]
    num_workers = 16
    chunk = B // num_workers
    out = _make_sc_kernel(B, N, chunk)(timesteps.reshape(B), noise_levels)
    return out.reshape(B, 1)


# speedup vs baseline: 1.0088x; 1.0067x over previous
import functools

import jax
import jax.numpy as jnp
from jax import lax
from jax.experimental import pallas as pl
from jax.experimental.pallas import tpu as pltpu
from jax.experimental.pallas import tpu_sc as plsc

_LANES = 16


def _make_sc_kernel(B, N, chunk):
    num_cores = 1
    mesh = plsc.VectorSubcoreMesh(
        core_axis_name="c",
        subcore_axis_name="s",
        num_cores=num_cores,
        num_subcores=8,
    )
    steps = chunk // _LANES

    @functools.partial(
        pl.kernel,
        mesh=mesh,
        out_type=jax.ShapeDtypeStruct((B,), jnp.float32),
        scratch_types=[
            pltpu.VMEM((chunk,), jnp.float32),
            pltpu.VMEM((N,), jnp.float32),
            pltpu.VMEM((chunk,), jnp.float32),
            pltpu.SemaphoreType.DMA,
            pltpu.SemaphoreType.DMA,
        ],
        compiler_params=pltpu.CompilerParams(needs_layout_passes=False),
    )
    def sc_kernel(ts_hbm, table_hbm, out_hbm, ts_v, tab_v, out_v, sem_a, sem_b):
        wid = lax.axis_index("s") * num_cores + lax.axis_index("c")
        base = wid * chunk
        cp_tab = pltpu.async_copy(table_hbm, tab_v, sem_a)
        cp_ts = pltpu.async_copy(ts_hbm.at[pl.ds(base, chunk)], ts_v, sem_b)
        cp_tab.wait()
        cp_ts.wait()

        @plsc.parallel_loop(0, steps, 1, unroll=8)
        def _(i):
            t = ts_v[pl.ds(i * _LANES, _LANES)]
            idx = t * jnp.float32(N - 1)
            lo = idx.astype(jnp.int32)
            w = idx - lo.astype(jnp.float32)
            hi = jnp.minimum(lo + 1, N - 1)
            lov = plsc.load_gather(tab_v, [lo])
            hiv = plsc.load_gather(tab_v, [hi])
            out_v[pl.ds(i * _LANES, _LANES)] = (1.0 - w) * lov + w * hiv

        pltpu.sync_copy(out_v, out_hbm.at[pl.ds(base, chunk)])

    return sc_kernel


@jax.jit
def kernel(timesteps, noise_levels):
    B = timesteps.shape[0]
    N = noise_levels.shape[0]
    num_workers = 8
    chunk = B // num_workers
    out = _make_sc_kernel(B, N, chunk)(timesteps.reshape(B), noise_levels)
    return out.reshape(B, 1)
